# baseline (speedup 1.0000x reference)
import functools

import jax
import jax.numpy as jnp
from jax.experimental import pallas as pl
from jax.experimental.pallas import tpu as pltpu

_B, _C, _H, _W = 256, 3, 224, 224
_E, _N = 16, 1000
_HW = _H * _W
_BLK_B = 32
_NSTEPS = _B // _BLK_B


def _moe_body(x_ref, Wg_ref, bg_ref, We_t_ref, be_ref, out_ref, w_ref,
              pooled_acc):
    i = pl.program_id(0)
    s = jnp.sum(x_ref[...], axis=2) * (1.0 / _HW)
    pooled_acc[pl.ds(i * _BLK_B, _BLK_B), :] = s

    @pl.when(i == _NSTEPS - 1)
    def _finalize():
        pooled = pooled_acc[...]
        logits = (jnp.dot(pooled, Wg_ref[...],
                          preferred_element_type=jnp.float32) + bg_ref[...])
        weights = jax.nn.softmax(logits, axis=1)
        w_ref[...] = weights
        m = jnp.max(weights, axis=1, keepdims=True)
        lane = jax.lax.broadcasted_iota(jnp.int32, (_B, _E), 1)
        eidx = jnp.min(jnp.where(weights == m, lane, _E), axis=1,
                       keepdims=True)
        onehot = (lane == eidx).astype(jnp.float32)
        acc = jnp.dot(onehot, be_ref[...],
                      preferred_element_type=jnp.float32)
        for c in range(_C):
            mp = onehot * pooled[:, c:c + 1]
            acc = acc + jnp.dot(mp, We_t_ref[c],
                                preferred_element_type=jnp.float32)
        out_ref[...] = acc


def kernel(x, Wg, bg, We, be):
    x3 = x.reshape(_B, _C, _HW)
    We_t = We.transpose(1, 0, 2)
    bg2 = bg.reshape(1, _E)
    out, weights = pl.pallas_call(
        _moe_body,
        grid=(_NSTEPS,),
        in_specs=[
            pl.BlockSpec((_BLK_B, _C, _HW), lambda i: (i, 0, 0)),
            pl.BlockSpec((_C, _E), lambda i: (0, 0)),
            pl.BlockSpec((1, _E), lambda i: (0, 0)),
            pl.BlockSpec((_C, _E, _N), lambda i: (0, 0, 0)),
            pl.BlockSpec((_E, _N), lambda i: (0, 0)),
        ],
        out_specs=[
            pl.BlockSpec((_B, _N), lambda i: (0, 0)),
            pl.BlockSpec((_B, _E), lambda i: (0, 0)),
        ],
        out_shape=[
            jax.ShapeDtypeStruct((_B, _N), jnp.float32),
            jax.ShapeDtypeStruct((_B, _E), jnp.float32),
        ],
        scratch_shapes=[pltpu.VMEM((_B, _C), jnp.float32)],
    )(x3, Wg, bg2, We_t, be)
    return (out, weights)


# speedup vs baseline: 1.5004x; 1.5004x over previous
import functools

import jax
import jax.numpy as jnp
from jax.experimental import pallas as pl
from jax.experimental.pallas import tpu as pltpu

_B, _C, _H, _W = 256, 3, 224, 224
_E, _N = 16, 1000
_HW = _H * _W
_HW_SUB = _HW // 128
_BLK_B = 16
_NSTEPS = _B // _BLK_B


def _moe_body(x_ref, Wg_ref, bg_ref, We_t_ref, be_ref, out_ref, w_ref,
              pooled_acc):
    i = pl.program_id(0)
    s = jnp.sum(x_ref[...], axis=(2, 3)) * (1.0 / _HW)
    pooled_acc[pl.ds(i * _BLK_B, _BLK_B), :] = s

    @pl.when(i == _NSTEPS - 1)
    def _finalize():
        pooled = pooled_acc[...]
        logits = (jnp.dot(pooled, Wg_ref[...],
                          preferred_element_type=jnp.float32) + bg_ref[...])
        weights = jax.nn.softmax(logits, axis=1)
        w_ref[...] = weights
        m = jnp.max(weights, axis=1, keepdims=True)
        lane = jax.lax.broadcasted_iota(jnp.int32, (_B, _E), 1)
        eidx = jnp.min(jnp.where(weights == m, lane, _E), axis=1,
                       keepdims=True)
        onehot = (lane == eidx).astype(jnp.float32)
        acc = jnp.dot(onehot, be_ref[...],
                      preferred_element_type=jnp.float32)
        for c in range(_C):
            mp = onehot * pooled[:, c:c + 1]
            acc = acc + jnp.dot(mp, We_t_ref[c],
                                preferred_element_type=jnp.float32)
        out_ref[...] = acc


def kernel(x, Wg, bg, We, be):
    x4 = x.reshape(_B, _C, _HW_SUB, 128)
    We_t = We.transpose(1, 0, 2)
    bg2 = bg.reshape(1, _E)
    out, weights = pl.pallas_call(
        _moe_body,
        grid=(_NSTEPS,),
        in_specs=[
            pl.BlockSpec((_BLK_B, _C, _HW_SUB, 128), lambda i: (i, 0, 0, 0)),
            pl.BlockSpec((_C, _E), lambda i: (0, 0)),
            pl.BlockSpec((1, _E), lambda i: (0, 0)),
            pl.BlockSpec((_C, _E, _N), lambda i: (0, 0, 0)),
            pl.BlockSpec((_E, _N), lambda i: (0, 0)),
        ],
        out_specs=[
            pl.BlockSpec((_B, _N), lambda i: (0, 0)),
            pl.BlockSpec((_B, _E), lambda i: (0, 0)),
        ],
        out_shape=[
            jax.ShapeDtypeStruct((_B, _N), jnp.float32),
            jax.ShapeDtypeStruct((_B, _E), jnp.float32),
        ],
        scratch_shapes=[pltpu.VMEM((_B, _C), jnp.float32)],
    )(x4, Wg, bg2, We_t, be)
    return (out, weights)
